# Initial kernel scaffold; baseline (speedup 1.0000x reference)
import jax
import jax.numpy as jnp
from jax.experimental import pallas as pl


def kernel(discriptors, flows_in, pos_2d):
    raise NotImplementedError("write your pallas kernel here")


# speedup vs baseline: 9.5032x; 9.5032x over previous
import functools

import jax
import jax.numpy as jnp
from jax import lax
from jax.experimental import pallas as pl
from jax.experimental.pallas import tpu as pltpu
from jax.experimental.pallas import tpu_sc as plsc

_EMBED = 256
_IMG = 224
_NC = 2
_NS = 16
_NW = _NC * _NS
_L = 16
_CHUNK = 128


def _sc_body(n_tok, fx_hbm, fy_hbm, desc_hbm, pos_hbm, out_hbm,
             fx_v, fy_v, idx_v, rows_v, desc_v, sem):
  b_per_w = n_tok // _NW
  n_chunks = b_per_w // _CHUNK
  wid = lax.axis_index("s") * _NC + lax.axis_index("c")
  w_base = wid * b_per_w

  def chunk_body(c, carry):
    base = w_base + c * _CHUNK
    pltpu.sync_copy(fx_hbm.at[pl.ds(base, _CHUNK)], fx_v)
    pltpu.sync_copy(fy_hbm.at[pl.ds(base, _CHUNK)], fy_v)
    for g in range(_CHUNK // _L):
      sl = pl.ds(g * _L, _L)
      xi = jnp.clip((fx_v[sl] * _IMG).astype(jnp.int32), 0, _IMG - 1)
      yi = jnp.clip((fy_v[sl] * _IMG).astype(jnp.int32), 0, _IMG - 1)
      idx_v[sl] = yi * _IMG + xi
    gather = pltpu.async_copy(pos_hbm.at[idx_v], rows_v, sem)
    pltpu.sync_copy(desc_hbm.at[pl.ds(base, _CHUNK)], desc_v)
    gather.wait()

    def add_row(r, carry2):
      for k in range(_EMBED // _L):
        sl = pl.ds(k * _L, _L)
        plsc.addupdate(desc_v.at[r, sl], rows_v[r, sl])
      return carry2

    lax.fori_loop(0, _CHUNK, add_row, 0)
    pltpu.sync_copy(desc_v, out_hbm.at[pl.ds(base, _CHUNK)])
    return carry

  lax.fori_loop(0, n_chunks, chunk_body, 0)


@jax.jit
def kernel(discriptors, flows_in, pos_2d):
  shape = discriptors.shape
  n_tok = shape[0] * shape[1]
  d = discriptors.reshape(n_tok, _EMBED)
  fx = flows_in[..., 0].reshape(n_tok)
  fy = flows_in[..., 1].reshape(n_tok)
  p = pos_2d.reshape(_IMG * _IMG, _EMBED)

  mesh = plsc.VectorSubcoreMesh(core_axis_name="c", subcore_axis_name="s")
  out = pl.kernel(
      functools.partial(_sc_body, n_tok),
      out_type=jax.ShapeDtypeStruct((n_tok, _EMBED), jnp.float32),
      mesh=mesh,
      scratch_types=[
          pltpu.VMEM((_CHUNK,), jnp.float32),
          pltpu.VMEM((_CHUNK,), jnp.float32),
          pltpu.VMEM((_CHUNK,), jnp.int32),
          pltpu.VMEM((_CHUNK, _EMBED), jnp.float32),
          pltpu.VMEM((_CHUNK, _EMBED), jnp.float32),
          pltpu.SemaphoreType.DMA,
      ],
  )(fx, fy, d, p)
  return out.reshape(shape)


# speedup vs baseline: 16.6812x; 1.7553x over previous
import functools

import jax
import jax.numpy as jnp
from jax import lax
from jax.experimental import pallas as pl
from jax.experimental.pallas import tpu as pltpu
from jax.experimental.pallas import tpu_sc as plsc

_EMBED = 256
_IMG = 224
_NC = 2
_NS = 16
_NW = _NC * _NS
_L = 16
_CHUNK = 64


def _sc_body(n_tok, fx_hbm, fy_hbm, desc_hbm, pos_hbm, out_hbm,
             fx_v, fy_v, idx_v, rows_v, desc_v, sem_g, sem_d):
  b_per_w = n_tok // _NW
  n_chunks = b_per_w // _CHUNK
  wid = lax.axis_index("s") * _NC + lax.axis_index("c")
  w_base = wid * b_per_w

  pltpu.sync_copy(fx_hbm.at[pl.ds(w_base, b_per_w)], fx_v)
  pltpu.sync_copy(fy_hbm.at[pl.ds(w_base, b_per_w)], fy_v)

  def compute_idx(i, carry):
    sl = pl.ds(i * _L, _L)
    xi = jnp.clip((fx_v[sl] * _IMG).astype(jnp.int32), 0, _IMG - 1)
    yi = jnp.clip((fy_v[sl] * _IMG).astype(jnp.int32), 0, _IMG - 1)
    idx_v[sl] = yi * _IMG + xi
    return carry

  lax.fori_loop(0, b_per_w // _L, compute_idx, 0)

  def start(c, b):
    pltpu.async_copy(pos_hbm.at[idx_v.at[pl.ds(c * _CHUNK, _CHUNK)]],
                     rows_v.at[b], sem_g.at[b])
    pltpu.async_copy(desc_hbm.at[pl.ds(w_base + c * _CHUNK, _CHUNK)],
                     desc_v.at[b], sem_d.at[b])

  def wait(b):
    pltpu.make_async_copy(desc_hbm.at[pl.ds(0, _CHUNK)],
                          rows_v.at[b], sem_g.at[b]).wait()
    pltpu.make_async_copy(desc_hbm.at[pl.ds(0, _CHUNK)],
                          desc_v.at[b], sem_d.at[b]).wait()

  start(0, 0)

  def group(g, carry):
    for b in range(2):
      c = g * 2 + b

      @pl.when(c + 1 < n_chunks)
      def _():
        start(c + 1, 1 - b)

      wait(b)

      def add_row(r, carry2):
        for k in range(_EMBED // _L):
          sl = pl.ds(k * _L, _L)
          plsc.addupdate(desc_v.at[b, r, sl], rows_v[b, r, sl])
        return carry2

      lax.fori_loop(0, _CHUNK, add_row, 0)
      pltpu.sync_copy(desc_v.at[b],
                      out_hbm.at[pl.ds(w_base + c * _CHUNK, _CHUNK)])
    return carry

  lax.fori_loop(0, n_chunks // 2, group, 0)


@jax.jit
def kernel(discriptors, flows_in, pos_2d):
  shape = discriptors.shape
  n_tok = shape[0] * shape[1]
  d = discriptors.reshape(n_tok, _EMBED)
  fx = flows_in[..., 0].reshape(n_tok)
  fy = flows_in[..., 1].reshape(n_tok)
  p = pos_2d.reshape(_IMG * _IMG, _EMBED)

  b_per_w = n_tok // _NW
  mesh = plsc.VectorSubcoreMesh(core_axis_name="c", subcore_axis_name="s")
  out = pl.kernel(
      functools.partial(_sc_body, n_tok),
      out_type=jax.ShapeDtypeStruct((n_tok, _EMBED), jnp.float32),
      mesh=mesh,
      scratch_types=[
          pltpu.VMEM((b_per_w,), jnp.float32),
          pltpu.VMEM((b_per_w,), jnp.float32),
          pltpu.VMEM((b_per_w,), jnp.int32),
          pltpu.VMEM((2, _CHUNK, _EMBED), jnp.float32),
          pltpu.VMEM((2, _CHUNK, _EMBED), jnp.float32),
          pltpu.SemaphoreType.DMA((2,)),
          pltpu.SemaphoreType.DMA((2,)),
      ],
  )(fx, fy, d, p)
  return out.reshape(shape)
